# baseline (speedup 1.0000x reference)
import functools

import jax
import jax.numpy as jnp
from jax import lax
from jax.experimental import pallas as pl
from jax.experimental.pallas import tpu as pltpu
from jax.experimental.pallas import tpu_sc as plsc

NUM_CORES = 2
NUM_SUBCORES = 16
NUM_WORKERS = NUM_CORES * NUM_SUBCORES
LANES = 16
BATCH = 16384
DIM = 32
BPW = BATCH // NUM_WORKERS
CHUNK = 128
NCHUNK = BPW // CHUNK


def _gmf_body(uidx_hbm, iidx_hbm, utabT_hbm, itabT_hbm, wb_hbm, out_hbm,
              uidx_v, iidx_v, uplane_v, iplane_v, wb_v, out_v, sem_u, sem_i):
    c = lax.axis_index("c")
    s = lax.axis_index("s")
    wid = s * NUM_CORES + c
    base = pl.multiple_of(wid * BPW, BPW)

    pltpu.sync_copy(wb_hbm, wb_v)
    for j in range(NCHUNK):
        pltpu.sync_copy(
            uidx_hbm.at[pl.ds(base + j * CHUNK, CHUNK)], uidx_v.at[j])
        pltpu.sync_copy(
            iidx_hbm.at[pl.ds(base + j * CHUNK, CHUNK)], iidx_v.at[j])

    copies = []
    for d in range(DIM):
        for j in range(NCHUNK):
            copies.append(pltpu.async_copy(
                utabT_hbm.at[d].at[uidx_v.at[j]],
                uplane_v.at[d, pl.ds(j * CHUNK, CHUNK)], sem_u))
            copies.append(pltpu.async_copy(
                itabT_hbm.at[d].at[iidx_v.at[j]],
                iplane_v.at[d, pl.ds(j * CHUNK, CHUNK)], sem_i))
    for cp in copies:
        cp.wait()

    bias_v = wb_v[DIM, :]

    def group_body(g, carry):
        off = pl.multiple_of(g * LANES, LANES)
        acc = bias_v
        for d in range(DIM):
            uv = uplane_v[d, pl.ds(off, LANES)]
            iv = iplane_v[d, pl.ds(off, LANES)]
            wv = wb_v[d, :]
            acc = acc + uv * iv * wv
        out_v[pl.ds(off, LANES)] = 1.0 / (1.0 + jnp.exp(-acc))
        return carry

    lax.fori_loop(0, BPW // LANES, group_body, 0)
    pltpu.sync_copy(out_v, out_hbm.at[pl.ds(base, BPW)])


@jax.jit
def _gmf_call(ui, ii, utabT, itabT, wb):
    mesh = plsc.VectorSubcoreMesh(core_axis_name="c", subcore_axis_name="s")
    f = functools.partial(
        pl.kernel,
        out_type=jax.ShapeDtypeStruct((BATCH,), jnp.float32),
        mesh=mesh,
        compiler_params=pltpu.CompilerParams(needs_layout_passes=False,
                                             use_tc_tiling_on_sc=False),
        scratch_types=[
            pltpu.VMEM((NCHUNK, CHUNK), jnp.int32),
            pltpu.VMEM((NCHUNK, CHUNK), jnp.int32),
            pltpu.VMEM((DIM, BPW), jnp.float32),
            pltpu.VMEM((DIM, BPW), jnp.float32),
            pltpu.VMEM((DIM + 1, LANES), jnp.float32),
            pltpu.VMEM((BPW,), jnp.float32),
            pltpu.SemaphoreType.DMA,
            pltpu.SemaphoreType.DMA,
        ],
    )(_gmf_body)
    return f(ui, ii, utabT, itabT, wb)


def kernel(user_indices, item_indices, user_table, item_table, affine_w, affine_b):
    ui = user_indices.astype(jnp.int32)
    ii = item_indices.astype(jnp.int32)
    wb = jnp.concatenate([
        jnp.broadcast_to(affine_w.reshape(DIM, 1), (DIM, LANES)),
        jnp.broadcast_to(affine_b.reshape(1, 1), (1, LANES)),
    ], axis=0).astype(jnp.float32)
    out = _gmf_call(ui, ii, user_table.T, item_table.T, wb)
    return out.reshape(BATCH, 1)
